# baseline (speedup 1.0000x reference)
import functools

import jax
import jax.numpy as jnp
from jax import lax
from jax.experimental import pallas as pl
from jax.experimental.pallas import tpu as pltpu
from jax.experimental.pallas import tpu_sc as plsc

LANES = 16
NC = 2
NS = 16
NW = NC * NS
D = 32
BLK = 256


def _mask_embed_body(xt_hbm, embf_hbm, ws_hbm, wi_hbm, wj_hbm, wk_hbm,
                     os_hbm, oi_hbm, oj_hbm, ok_hbm,
                     idx_v, idxm_v, emb_v, msv, miv, mjv, mkv,
                     out_s, out_i, out_j, out_k, gsem, ssem,
                     fields, batch):
    wid = lax.axis_index("s") * NC + lax.axis_index("c")
    nblk = batch // BLK
    nitems = fields * nblk // NW

    def item_body(t, carry):
        item = wid + t * NW
        f = item // nblk
        blk = item % nblk
        b0 = blk * BLK

        pltpu.sync_copy(xt_hbm.at[f, pl.ds(b0, BLK)], idx_v)

        for c in range(D):
            for g in range(BLK // LANES):
                sl = pl.ds(g * LANES, LANES)
                idxm_v[pl.ds(c * BLK + g * LANES, LANES)] = idx_v[sl] + c * 1000000

        cps = [
            pltpu.async_copy(ws_hbm.at[idx_v], msv, gsem),
            pltpu.async_copy(wi_hbm.at[idx_v], miv, gsem),
            pltpu.async_copy(wj_hbm.at[idx_v], mjv, gsem),
            pltpu.async_copy(wk_hbm.at[idx_v], mkv, gsem),
            pltpu.async_copy(embf_hbm.at[idxm_v], emb_v, gsem),
        ]
        for cp in cps:
            cp.wait()

        for g in range(BLK // LANES):
            sl = pl.ds(g * LANES, LANES)
            ms = 2.0 / (1.0 + jnp.exp(-msv[sl]))
            mi = 2.0 / (1.0 + jnp.exp(-miv[sl]))
            mj = 2.0 / (1.0 + jnp.exp(-mjv[sl]))
            mk = 2.0 / (1.0 + jnp.exp(-mkv[sl]))
            sel = ms > 1.0
            msv[sl] = ms
            miv[sl] = jnp.where(sel, ms, mi)
            mjv[sl] = jnp.where(sel, ms, mj)
            mkv[sl] = jnp.where(sel, ms, mk)

        def mul_body(g, carry2):
            sl = pl.ds(g * LANES, LANES)
            ms = msv[sl]
            mi = miv[sl]
            mj = mjv[sl]
            mk = mkv[sl]
            for c in range(D):
                e = emb_v[pl.ds(c * BLK + g * LANES, LANES)]
                out_s[c, sl] = e * ms
                out_i[c, sl] = e * mi
                out_j[c, sl] = e * mj
                out_k[c, sl] = e * mk
            return carry2

        lax.fori_loop(0, BLK // LANES, mul_body, 0)

        r0 = f * D
        wps = [
            pltpu.async_copy(out_s, os_hbm.at[pl.ds(r0, D), pl.ds(b0, BLK)], ssem),
            pltpu.async_copy(out_i, oi_hbm.at[pl.ds(r0, D), pl.ds(b0, BLK)], ssem),
            pltpu.async_copy(out_j, oj_hbm.at[pl.ds(r0, D), pl.ds(b0, BLK)], ssem),
            pltpu.async_copy(out_k, ok_hbm.at[pl.ds(r0, D), pl.ds(b0, BLK)], ssem),
        ]
        for wp in wps:
            wp.wait()
        return carry

    lax.fori_loop(0, nitems, item_body, 0)


def kernel(x, embedding, mask_weight_s, mask_weight_i, mask_weight_j, mask_weight_k):
    b, f = x.shape
    assert (f * (b // BLK)) % NW == 0, (b, f)

    xt = x.T.astype(jnp.int32)
    embf = embedding.T.reshape(-1)
    ws = mask_weight_s.reshape(-1)
    wi = mask_weight_i.reshape(-1)
    wj = mask_weight_j.reshape(-1)
    wk = mask_weight_k.reshape(-1)

    out_sds = jax.ShapeDtypeStruct((f * D, b), jnp.float32)
    run = pl.kernel(
        functools.partial(_mask_embed_body, fields=f, batch=b),
        out_type=(out_sds, out_sds, out_sds, out_sds),
        mesh=plsc.VectorSubcoreMesh(core_axis_name="c", subcore_axis_name="s"),
        scratch_types=[
            pltpu.VMEM((BLK,), jnp.int32),
            pltpu.VMEM((D * BLK,), jnp.int32),
            pltpu.VMEM((D * BLK,), jnp.float32),
            pltpu.VMEM((BLK,), jnp.float32),
            pltpu.VMEM((BLK,), jnp.float32),
            pltpu.VMEM((BLK,), jnp.float32),
            pltpu.VMEM((BLK,), jnp.float32),
            pltpu.VMEM((D, BLK), jnp.float32),
            pltpu.VMEM((D, BLK), jnp.float32),
            pltpu.VMEM((D, BLK), jnp.float32),
            pltpu.VMEM((D, BLK), jnp.float32),
            pltpu.SemaphoreType.DMA,
            pltpu.SemaphoreType.DMA,
        ],
    )
    o_s, o_i, o_j, o_k = run(xt, embf, ws, wi, wj, wk)

    def untranspose(o):
        return o.reshape(f, D, b).transpose(2, 0, 1)

    return (untranspose(o_s), untranspose(o_i),
            untranspose(o_j), untranspose(o_k))


# speedup vs baseline: 1.3740x; 1.3740x over previous
import functools

import jax
import jax.numpy as jnp
from jax import lax
from jax.experimental import pallas as pl
from jax.experimental.pallas import tpu as pltpu
from jax.experimental.pallas import tpu_sc as plsc

LANES = 16
NC = 2
NS = 16
NW = NC * NS
D = 32
BLK = 256


def _mask_embed_body(xt_hbm, embf_hbm, ws_hbm, wi_hbm, wj_hbm, wk_hbm,
                     os_hbm, oi_hbm, oj_hbm, ok_hbm,
                     idx_v, idxm_v, emb_v, msv, miv, mjv, mkv,
                     out_s, out_i, out_j, out_k, gsem, ssem,
                     fields, batch):
    wid = lax.axis_index("s") * NC + lax.axis_index("c")
    nblk = batch // BLK
    nitems = fields * nblk // NW

    def item_body(t, carry):
        item = wid + t * NW
        f = item // nblk
        blk = item % nblk
        b0 = blk * BLK

        pltpu.sync_copy(xt_hbm.at[f, pl.ds(b0, BLK)], idx_v)

        for c in range(D):
            for g in range(BLK // LANES):
                sl = pl.ds(g * LANES, LANES)
                idxm_v[pl.ds(c * BLK + g * LANES, LANES)] = idx_v[sl] + c * 1000000

        cps = [
            pltpu.async_copy(ws_hbm.at[idx_v], msv, gsem),
            pltpu.async_copy(wi_hbm.at[idx_v], miv, gsem),
            pltpu.async_copy(wj_hbm.at[idx_v], mjv, gsem),
            pltpu.async_copy(wk_hbm.at[idx_v], mkv, gsem),
            pltpu.async_copy(embf_hbm.at[idxm_v], emb_v, gsem),
        ]
        for cp in cps:
            cp.wait()

        for g in range(BLK // LANES):
            sl = pl.ds(g * LANES, LANES)
            ms = 2.0 / (1.0 + jnp.exp(-msv[sl]))
            mi = 2.0 / (1.0 + jnp.exp(-miv[sl]))
            mj = 2.0 / (1.0 + jnp.exp(-mjv[sl]))
            mk = 2.0 / (1.0 + jnp.exp(-mkv[sl]))
            sel = ms > 1.0
            msv[sl] = ms
            miv[sl] = jnp.where(sel, ms, mi)
            mjv[sl] = jnp.where(sel, ms, mj)
            mkv[sl] = jnp.where(sel, ms, mk)

        def mul_body(g, carry2):
            sl = pl.ds(g * LANES, LANES)
            ms = msv[sl]
            mi = miv[sl]
            mj = mjv[sl]
            mk = mkv[sl]
            for c in range(D):
                e = emb_v[pl.ds(c * BLK + g * LANES, LANES)]
                out_s[c, sl] = e * ms
                out_i[c, sl] = e * mi
                out_j[c, sl] = e * mj
                out_k[c, sl] = e * mk
            return carry2

        lax.fori_loop(0, BLK // LANES, mul_body, 0)

        r0 = f * D
        wps = [
            pltpu.async_copy(out_s, os_hbm.at[pl.ds(r0, D), pl.ds(b0, BLK)], ssem),
            pltpu.async_copy(out_i, oi_hbm.at[pl.ds(r0, D), pl.ds(b0, BLK)], ssem),
            pltpu.async_copy(out_j, oj_hbm.at[pl.ds(r0, D), pl.ds(b0, BLK)], ssem),
            pltpu.async_copy(out_k, ok_hbm.at[pl.ds(r0, D), pl.ds(b0, BLK)], ssem),
        ]
        for wp in wps:
            wp.wait()
        return carry

    lax.fori_loop(0, nitems, item_body, 0)


def kernel(x, embedding, mask_weight_s, mask_weight_i, mask_weight_j, mask_weight_k):
    b, f = x.shape
    assert (f * (b // BLK)) % NW == 0, (b, f)

    xt = x.T.astype(jnp.int32)
    embf = jnp.concatenate([embedding[:, c] for c in range(D)])
    ws = mask_weight_s.reshape(-1)
    wi = mask_weight_i.reshape(-1)
    wj = mask_weight_j.reshape(-1)
    wk = mask_weight_k.reshape(-1)

    out_sds = jax.ShapeDtypeStruct((f * D, b), jnp.float32)
    run = pl.kernel(
        functools.partial(_mask_embed_body, fields=f, batch=b),
        out_type=(out_sds, out_sds, out_sds, out_sds),
        mesh=plsc.VectorSubcoreMesh(core_axis_name="c", subcore_axis_name="s"),
        scratch_types=[
            pltpu.VMEM((BLK,), jnp.int32),
            pltpu.VMEM((D * BLK,), jnp.int32),
            pltpu.VMEM((D * BLK,), jnp.float32),
            pltpu.VMEM((BLK,), jnp.float32),
            pltpu.VMEM((BLK,), jnp.float32),
            pltpu.VMEM((BLK,), jnp.float32),
            pltpu.VMEM((BLK,), jnp.float32),
            pltpu.VMEM((D, BLK), jnp.float32),
            pltpu.VMEM((D, BLK), jnp.float32),
            pltpu.VMEM((D, BLK), jnp.float32),
            pltpu.VMEM((D, BLK), jnp.float32),
            pltpu.SemaphoreType.DMA,
            pltpu.SemaphoreType.DMA,
        ],
    )
    o_s, o_i, o_j, o_k = run(xt, embf, ws, wi, wj, wk)

    def untranspose(o):
        return o.reshape(f, D, b).transpose(2, 0, 1)

    return (untranspose(o_s), untranspose(o_i),
            untranspose(o_j), untranspose(o_k))


# speedup vs baseline: 6.8809x; 5.0079x over previous
import functools

import jax
import jax.numpy as jnp
from jax import lax
from jax.experimental import pallas as pl
from jax.experimental.pallas import tpu as pltpu
from jax.experimental.pallas import tpu_sc as plsc

LANES = 16
NC = 2
NS = 16
NW = NC * NS
D = 32
BLK = 256


def _detile_body(embt_hbm, flat_hbm, buf0, buf1, tail2d, tail1d, rsem, wsem, v):
    wid = lax.axis_index("s") * NC + lax.axis_index("c")
    blocks = [(k * 32000, 32000) for k in range(31)] + [(992000, 7936)]
    bufs = (buf0, buf1)
    wcps = []
    for k, (r0, w) in enumerate(blocks):
        buf = bufs[k % 2]
        if k >= 2:
            wcps[k - 2].wait()
        pltpu.async_copy(embt_hbm.at[wid, pl.ds(r0, w)],
                         buf.at[pl.ds(0, w)], rsem).wait()
        wcps.append(pltpu.async_copy(
            buf.at[pl.ds(0, w)], flat_hbm.at[pl.ds(wid * v + r0, w)], wsem))
    tail = v - v % 128
    pltpu.async_copy(embt_hbm.at[pl.ds(wid, 1), pl.ds(tail, v % 128)],
                     tail2d, rsem).wait()
    for k in range((v % 128) // LANES):
        sl = pl.ds(k * LANES, LANES)
        tail1d[sl] = tail2d[0, sl]
    wcps.append(pltpu.async_copy(
        tail1d, flat_hbm.at[pl.ds(wid * v + tail, v % 128)], wsem))
    for cp in wcps[-3:]:
        cp.wait()


def _mask_embed_body(xt_hbm, embf_hbm, ws_hbm, wi_hbm, wj_hbm, wk_hbm,
                     os_hbm, oi_hbm, oj_hbm, ok_hbm,
                     idx_v, idxm_v, emb_v, msv, miv, mjv, mkv,
                     out_s, out_i, out_j, out_k, gsem, ssem,
                     fields, batch):
    wid = lax.axis_index("s") * NC + lax.axis_index("c")
    nblk = batch // BLK
    nitems = fields * nblk // NW

    def item_body(t, carry):
        item = wid + t * NW
        f = item // nblk
        blk = item % nblk
        b0 = blk * BLK

        pltpu.sync_copy(xt_hbm.at[f, pl.ds(b0, BLK)], idx_v)

        for c in range(D):
            for g in range(BLK // LANES):
                sl = pl.ds(g * LANES, LANES)
                idxm_v[pl.ds(c * BLK + g * LANES, LANES)] = idx_v[sl] + c * 1000000

        cps = [
            pltpu.async_copy(ws_hbm.at[idx_v], msv, gsem),
            pltpu.async_copy(wi_hbm.at[idx_v], miv, gsem),
            pltpu.async_copy(wj_hbm.at[idx_v], mjv, gsem),
            pltpu.async_copy(wk_hbm.at[idx_v], mkv, gsem),
            pltpu.async_copy(embf_hbm.at[idxm_v], emb_v, gsem),
        ]
        for cp in cps:
            cp.wait()

        for g in range(BLK // LANES):
            sl = pl.ds(g * LANES, LANES)
            ms = 2.0 / (1.0 + jnp.exp(-msv[sl]))
            mi = 2.0 / (1.0 + jnp.exp(-miv[sl]))
            mj = 2.0 / (1.0 + jnp.exp(-mjv[sl]))
            mk = 2.0 / (1.0 + jnp.exp(-mkv[sl]))
            sel = ms > 1.0
            msv[sl] = ms
            miv[sl] = jnp.where(sel, ms, mi)
            mjv[sl] = jnp.where(sel, ms, mj)
            mkv[sl] = jnp.where(sel, ms, mk)

        def mul_body(g, carry2):
            sl = pl.ds(g * LANES, LANES)
            ms = msv[sl]
            mi = miv[sl]
            mj = mjv[sl]
            mk = mkv[sl]
            for c in range(D):
                e = emb_v[pl.ds(c * BLK + g * LANES, LANES)]
                out_s[c, sl] = e * ms
                out_i[c, sl] = e * mi
                out_j[c, sl] = e * mj
                out_k[c, sl] = e * mk
            return carry2

        lax.fori_loop(0, BLK // LANES, mul_body, 0)

        r0 = f * D
        wps = [
            pltpu.async_copy(out_s, os_hbm.at[pl.ds(r0, D), pl.ds(b0, BLK)], ssem),
            pltpu.async_copy(out_i, oi_hbm.at[pl.ds(r0, D), pl.ds(b0, BLK)], ssem),
            pltpu.async_copy(out_j, oj_hbm.at[pl.ds(r0, D), pl.ds(b0, BLK)], ssem),
            pltpu.async_copy(out_k, ok_hbm.at[pl.ds(r0, D), pl.ds(b0, BLK)], ssem),
        ]
        for wp in wps:
            wp.wait()
        return carry

    lax.fori_loop(0, nitems, item_body, 0)


def kernel(x, embedding, mask_weight_s, mask_weight_i, mask_weight_j, mask_weight_k):
    b, f = x.shape
    assert (f * (b // BLK)) % NW == 0, (b, f)

    xt = x.T.astype(jnp.int32)
    v = embedding.shape[0]
    embt = embedding.T
    detile = pl.kernel(
        functools.partial(_detile_body, v=v),
        out_type=jax.ShapeDtypeStruct((D * v,), jnp.float32),
        mesh=plsc.VectorSubcoreMesh(core_axis_name="c", subcore_axis_name="s"),
        scratch_types=[
            pltpu.VMEM((32000,), jnp.float32),
            pltpu.VMEM((32000,), jnp.float32),
            pltpu.VMEM((1, v % 128), jnp.float32),
            pltpu.VMEM((v % 128,), jnp.float32),
            pltpu.SemaphoreType.DMA,
            pltpu.SemaphoreType.DMA,
        ],
    )
    embf = detile(embt)
    ws = mask_weight_s.reshape(-1)
    wi = mask_weight_i.reshape(-1)
    wj = mask_weight_j.reshape(-1)
    wk = mask_weight_k.reshape(-1)

    out_sds = jax.ShapeDtypeStruct((f * D, b), jnp.float32)
    run = pl.kernel(
        functools.partial(_mask_embed_body, fields=f, batch=b),
        out_type=(out_sds, out_sds, out_sds, out_sds),
        mesh=plsc.VectorSubcoreMesh(core_axis_name="c", subcore_axis_name="s"),
        scratch_types=[
            pltpu.VMEM((BLK,), jnp.int32),
            pltpu.VMEM((D * BLK,), jnp.int32),
            pltpu.VMEM((D * BLK,), jnp.float32),
            pltpu.VMEM((BLK,), jnp.float32),
            pltpu.VMEM((BLK,), jnp.float32),
            pltpu.VMEM((BLK,), jnp.float32),
            pltpu.VMEM((BLK,), jnp.float32),
            pltpu.VMEM((D, BLK), jnp.float32),
            pltpu.VMEM((D, BLK), jnp.float32),
            pltpu.VMEM((D, BLK), jnp.float32),
            pltpu.VMEM((D, BLK), jnp.float32),
            pltpu.SemaphoreType.DMA,
            pltpu.SemaphoreType.DMA,
        ],
    )
    o_s, o_i, o_j, o_k = run(xt, embf, ws, wi, wj, wk)

    def untranspose(o):
        return o.reshape(f, D, b).transpose(2, 0, 1)

    return (untranspose(o_s), untranspose(o_i),
            untranspose(o_j), untranspose(o_k))


# speedup vs baseline: 7.2761x; 1.0574x over previous
import functools

import jax
import jax.numpy as jnp
from jax import lax
from jax.experimental import pallas as pl
from jax.experimental.pallas import tpu as pltpu
from jax.experimental.pallas import tpu_sc as plsc

LANES = 16
NC = 2
NS = 16
NW = NC * NS
D = 32
BLK = 256


def _detile_body(embt_hbm, flat_hbm, buf0, buf1, tail2d, tail1d, rsem, wsem, v):
    wid = lax.axis_index("s") * NC + lax.axis_index("c")
    blocks = [(k * 32000, 32000) for k in range(31)] + [(992000, 7936)]
    bufs = (buf0, buf1)
    wcps = []
    for k, (r0, w) in enumerate(blocks):
        buf = bufs[k % 2]
        if k >= 2:
            wcps[k - 2].wait()
        pltpu.async_copy(embt_hbm.at[wid, pl.ds(r0, w)],
                         buf.at[pl.ds(0, w)], rsem).wait()
        wcps.append(pltpu.async_copy(
            buf.at[pl.ds(0, w)], flat_hbm.at[pl.ds(wid * v + r0, w)], wsem))
    tail = v - v % 128
    pltpu.async_copy(embt_hbm.at[pl.ds(wid, 1), pl.ds(tail, v % 128)],
                     tail2d, rsem).wait()
    for k in range((v % 128) // LANES):
        sl = pl.ds(k * LANES, LANES)
        tail1d[sl] = tail2d[0, sl]
    wcps.append(pltpu.async_copy(
        tail1d, flat_hbm.at[pl.ds(wid * v + tail, v % 128)], wsem))
    for cp in wcps[-3:]:
        cp.wait()


class _Bufs:
    def __init__(self, idx, idxm, emb, w, out, gsem, wsem):
        self.idx, self.idxm, self.emb = idx, idxm, emb
        self.w, self.out, self.gsem, self.wsem = w, out, gsem, wsem


def _mask_embed_body(xt_hbm, embf_hbm, ws_hbm, wi_hbm, wj_hbm, wk_hbm,
                     os_hbm, oi_hbm, oj_hbm, ok_hbm, *scratch,
                     fields, batch):
    (idx_a, idx_b, idxm_a, idxm_b, emb_a, emb_b,
     wsa, wia, wja, wka, wsb, wib, wjb, wkb,
     osa, oia, oja, oka, osb, oib, ojb, okb,
     gsem_a, gsem_b, wsem_a, wsem_b) = scratch
    wid = lax.axis_index("s") * NC + lax.axis_index("c")
    nblk = batch // BLK
    nitems = fields * nblk // NW
    assert nitems % 2 == 1

    A = _Bufs(idx_a, idxm_a, emb_a, (wsa, wia, wja, wka),
              (osa, oia, oja, oka), gsem_a, wsem_a)
    B = _Bufs(idx_b, idxm_b, emb_b, (wsb, wib, wjb, wkb),
              (osb, oib, ojb, okb), gsem_b, wsem_b)
    ohbm = (os_hbm, oi_hbm, oj_hbm, ok_hbm)
    whbm = (ws_hbm, wi_hbm, wj_hbm, wk_hbm)

    def fb(t):
        item = wid + t * NW
        return item // nblk, (item % nblk) * BLK

    def fire(t, s):
        f, b0 = fb(t)
        pltpu.sync_copy(xt_hbm.at[f, pl.ds(b0, BLK)], s.idx)

        def cbody(c, cr):
            base = c * BLK
            for g in range(BLK // LANES):
                sl = pl.ds(g * LANES, LANES)
                s.idxm[pl.ds(base + g * LANES, LANES)] = s.idx[sl] + c * 1000000
            return cr

        lax.fori_loop(0, D, cbody, 0)
        for q in range(4):
            pltpu.async_copy(whbm[q].at[s.idx], s.w[q], s.gsem)
        pltpu.async_copy(embf_hbm.at[s.idxm], s.emb, s.gsem)

    def drain_gathers(s):
        for q in range(4):
            pltpu.make_async_copy(whbm[q].at[s.idx], s.w[q], s.gsem).wait()
        pltpu.make_async_copy(embf_hbm.at[s.idxm], s.emb, s.gsem).wait()

    def compute(s):
        for g in range(BLK // LANES):
            sl = pl.ds(g * LANES, LANES)
            ms = 2.0 / (1.0 + jnp.exp(-s.w[0][sl]))
            mi = 2.0 / (1.0 + jnp.exp(-s.w[1][sl]))
            mj = 2.0 / (1.0 + jnp.exp(-s.w[2][sl]))
            mk = 2.0 / (1.0 + jnp.exp(-s.w[3][sl]))
            sel = ms > 1.0
            s.w[0][sl] = ms
            s.w[1][sl] = jnp.where(sel, ms, mi)
            s.w[2][sl] = jnp.where(sel, ms, mj)
            s.w[3][sl] = jnp.where(sel, ms, mk)

        def mul_body(g, carry2):
            sl = pl.ds(g * LANES, LANES)
            m = [s.w[q][sl] for q in range(4)]
            for c in range(D):
                e = s.emb[pl.ds(c * BLK + g * LANES, LANES)]
                for q in range(4):
                    s.out[q][c, sl] = e * m[q]
            return carry2

        lax.fori_loop(0, BLK // LANES, mul_body, 0)

    def writeback(t, s):
        f, b0 = fb(t)
        for q in range(4):
            pltpu.async_copy(
                s.out[q], ohbm[q].at[pl.ds(f * D, D), pl.ds(b0, BLK)], s.wsem)

    def drain_writes(s):
        for q in range(4):
            pltpu.make_async_copy(
                s.out[q], ohbm[q].at[pl.ds(0, D), pl.ds(0, BLK)], s.wsem).wait()

    fire(0, A)

    def pair(p, cr):
        t0 = 2 * p
        drain_gathers(A)
        fire(t0 + 1, B)

        @pl.when(p > 0)
        def _():
            drain_writes(A)

        compute(A)
        writeback(t0, A)
        drain_gathers(B)
        fire(t0 + 2, A)

        @pl.when(p > 0)
        def _():
            drain_writes(B)

        compute(B)
        writeback(t0 + 1, B)
        return cr

    lax.fori_loop(0, (nitems - 1) // 2, pair, 0)

    drain_gathers(A)
    drain_writes(A)
    compute(A)
    writeback(nitems - 1, A)
    drain_writes(A)
    drain_writes(B)


def kernel(x, embedding, mask_weight_s, mask_weight_i, mask_weight_j, mask_weight_k):
    b, f = x.shape
    assert (f * (b // BLK)) % NW == 0, (b, f)

    xt = x.T.astype(jnp.int32)
    v = embedding.shape[0]
    embt = embedding.T
    detile = pl.kernel(
        functools.partial(_detile_body, v=v),
        out_type=jax.ShapeDtypeStruct((D * v,), jnp.float32),
        mesh=plsc.VectorSubcoreMesh(core_axis_name="c", subcore_axis_name="s"),
        scratch_types=[
            pltpu.VMEM((32000,), jnp.float32),
            pltpu.VMEM((32000,), jnp.float32),
            pltpu.VMEM((1, v % 128), jnp.float32),
            pltpu.VMEM((v % 128,), jnp.float32),
            pltpu.SemaphoreType.DMA,
            pltpu.SemaphoreType.DMA,
        ],
    )
    embf = detile(embt)
    ws = mask_weight_s.reshape(-1)
    wi = mask_weight_i.reshape(-1)
    wj = mask_weight_j.reshape(-1)
    wk = mask_weight_k.reshape(-1)

    out_sds = jax.ShapeDtypeStruct((f * D, b), jnp.float32)
    run = pl.kernel(
        functools.partial(_mask_embed_body, fields=f, batch=b),
        out_type=(out_sds, out_sds, out_sds, out_sds),
        mesh=plsc.VectorSubcoreMesh(core_axis_name="c", subcore_axis_name="s"),
        scratch_types=(
            [pltpu.VMEM((BLK,), jnp.int32)] * 2
            + [pltpu.VMEM((D * BLK,), jnp.int32)] * 2
            + [pltpu.VMEM((D * BLK,), jnp.float32)] * 2
            + [pltpu.VMEM((BLK,), jnp.float32)] * 8
            + [pltpu.VMEM((D, BLK), jnp.float32)] * 8
            + [pltpu.SemaphoreType.DMA] * 4
        ),
    )
    o_s, o_i, o_j, o_k = run(xt, embf, ws, wi, wj, wk)

    def untranspose(o):
        return o.reshape(f, D, b).transpose(2, 0, 1)

    return (untranspose(o_s), untranspose(o_i),
            untranspose(o_j), untranspose(o_k))


# speedup vs baseline: 7.2774x; 1.0002x over previous
import functools

import jax
import jax.numpy as jnp
from jax import lax
from jax.experimental import pallas as pl
from jax.experimental.pallas import tpu as pltpu
from jax.experimental.pallas import tpu_sc as plsc

LANES = 16
NC = 2
NS = 16
NW = NC * NS
D = 32
BLK = 256


def _detile_body(embt_hbm, flat_hbm, buf0, buf1, tail2d, tail1d, rsem, wsem, v):
    wid = lax.axis_index("s") * NC + lax.axis_index("c")
    blocks = [(k * 32000, 32000) for k in range(31)] + [(992000, 7936)]
    bufs = (buf0, buf1)
    wcps = []
    for k, (r0, w) in enumerate(blocks):
        buf = bufs[k % 2]
        if k >= 2:
            wcps[k - 2].wait()
        pltpu.async_copy(embt_hbm.at[wid, pl.ds(r0, w)],
                         buf.at[pl.ds(0, w)], rsem).wait()
        wcps.append(pltpu.async_copy(
            buf.at[pl.ds(0, w)], flat_hbm.at[pl.ds(wid * v + r0, w)], wsem))
    tail = v - v % 128
    pltpu.async_copy(embt_hbm.at[pl.ds(wid, 1), pl.ds(tail, v % 128)],
                     tail2d, rsem).wait()
    for k in range((v % 128) // LANES):
        sl = pl.ds(k * LANES, LANES)
        tail1d[sl] = tail2d[0, sl]
    wcps.append(pltpu.async_copy(
        tail1d, flat_hbm.at[pl.ds(wid * v + tail, v % 128)], wsem))
    for cp in wcps[-3:]:
        cp.wait()


class _Bufs:
    def __init__(self, idx, idxm, emb, w, out, gsem, wsem):
        self.idx, self.idxm, self.emb = idx, idxm, emb
        self.w, self.out, self.gsem, self.wsem = w, out, gsem, wsem


def _mask_embed_body(xt_hbm, embf_hbm, ws_hbm, wi_hbm, wj_hbm, wk_hbm,
                     os_hbm, oi_hbm, oj_hbm, ok_hbm, *scratch,
                     fields, batch):
    (idx_a, idx_b, idxm_a, idxm_b, emb_a, emb_b,
     wsa, wia, wja, wka, wsb, wib, wjb, wkb,
     osa, oia, oja, oka, osb, oib, ojb, okb,
     gsem_a, gsem_b, wsem_a, wsem_b) = scratch
    wid = lax.axis_index("s") * NC + lax.axis_index("c")
    nblk = batch // BLK
    nitems = fields * nblk // NW
    assert nitems % 2 == 1

    A = _Bufs(idx_a, idxm_a, emb_a, (wsa, wia, wja, wka),
              (osa, oia, oja, oka), gsem_a, wsem_a)
    B = _Bufs(idx_b, idxm_b, emb_b, (wsb, wib, wjb, wkb),
              (osb, oib, ojb, okb), gsem_b, wsem_b)
    ohbm = (os_hbm, oi_hbm, oj_hbm, ok_hbm)
    whbm = (ws_hbm, wi_hbm, wj_hbm, wk_hbm)

    def fb(t):
        item = wid + t * NW
        return item // nblk, (item % nblk) * BLK

    def fire(t, s):
        f, b0 = fb(t)
        pltpu.sync_copy(xt_hbm.at[f, pl.ds(b0, BLK)], s.idx)

        def cbody(c, cr):
            base = c * BLK
            for g in range(BLK // LANES):
                sl = pl.ds(g * LANES, LANES)
                s.idxm[pl.ds(base + g * LANES, LANES)] = s.idx[sl] + c * 1000000
            return cr

        lax.fori_loop(0, D, cbody, 0)
        for q in range(4):
            pltpu.async_copy(whbm[q].at[s.idx], s.w[q], s.gsem)
        pltpu.async_copy(embf_hbm.at[s.idxm], s.emb, s.gsem)

    def drain_gathers(s):
        for q in range(4):
            pltpu.make_async_copy(whbm[q].at[s.idx], s.w[q], s.gsem).wait()
        pltpu.make_async_copy(embf_hbm.at[s.idxm], s.emb, s.gsem).wait()

    def compute(s):
        for g in range(BLK // LANES):
            sl = pl.ds(g * LANES, LANES)
            ms = 2.0 / (1.0 + jnp.exp(-s.w[0][sl]))
            mi = 2.0 / (1.0 + jnp.exp(-s.w[1][sl]))
            mj = 2.0 / (1.0 + jnp.exp(-s.w[2][sl]))
            mk = 2.0 / (1.0 + jnp.exp(-s.w[3][sl]))
            sel = ms > 1.0
            s.w[0][sl] = ms
            s.w[1][sl] = jnp.where(sel, ms, mi)
            s.w[2][sl] = jnp.where(sel, ms, mj)
            s.w[3][sl] = jnp.where(sel, ms, mk)

        def mul_body(g, carry2):
            sl = pl.ds(g * LANES, LANES)
            m = [s.w[q][sl] for q in range(4)]
            for c in range(D):
                e = s.emb[pl.ds(c * BLK + g * LANES, LANES)]
                for q in range(4):
                    s.out[q][c, sl] = e * m[q]
            return carry2

        lax.fori_loop(0, BLK // LANES, mul_body, 0)

    def writeback(t, s):
        f, b0 = fb(t)
        for q in range(4):
            pltpu.async_copy(
                s.out[q], ohbm[q].at[pl.ds(f * D, D), pl.ds(b0, BLK)], s.wsem)

    def drain_writes(s):
        for q in range(4):
            pltpu.make_async_copy(
                s.out[q], ohbm[q].at[pl.ds(0, D), pl.ds(0, BLK)], s.wsem).wait()

    fire(0, A)

    def pair(p, cr):
        t0 = 2 * p
        drain_gathers(A)
        fire(t0 + 1, B)

        @pl.when(p > 0)
        def _():
            drain_writes(A)

        compute(A)
        writeback(t0, A)
        drain_gathers(B)
        fire(t0 + 2, A)

        @pl.when(p > 0)
        def _():
            drain_writes(B)

        compute(B)
        writeback(t0 + 1, B)
        return cr

    lax.fori_loop(0, (nitems - 1) // 2, pair, 0)

    drain_gathers(A)
    drain_writes(A)
    compute(A)
    writeback(nitems - 1, A)
    drain_writes(A)
    drain_writes(B)


def kernel(x, embedding, mask_weight_s, mask_weight_i, mask_weight_j, mask_weight_k):
    b, f = x.shape
    assert (f * (b // BLK)) % NW == 0, (b, f)

    xt = x.T.astype(jnp.int32)
    v = embedding.shape[0]
    embt = embedding.T
    detile = pl.kernel(
        functools.partial(_detile_body, v=v),
        out_type=jax.ShapeDtypeStruct((D * v,), jnp.float32),
        mesh=plsc.VectorSubcoreMesh(core_axis_name="c", subcore_axis_name="s"),
        scratch_types=[
            pltpu.VMEM((32000,), jnp.float32),
            pltpu.VMEM((32000,), jnp.float32),
            pltpu.VMEM((1, v % 128), jnp.float32),
            pltpu.VMEM((v % 128,), jnp.float32),
            pltpu.SemaphoreType.DMA,
            pltpu.SemaphoreType.DMA,
        ],
    )
    embf = detile(embt)
    ws = mask_weight_s[:, 0]
    wi = mask_weight_i[:, 0]
    wj = mask_weight_j[:, 0]
    wk = mask_weight_k[:, 0]

    out_sds = jax.ShapeDtypeStruct((f * D, b), jnp.float32)
    run = pl.kernel(
        functools.partial(_mask_embed_body, fields=f, batch=b),
        out_type=(out_sds, out_sds, out_sds, out_sds),
        mesh=plsc.VectorSubcoreMesh(core_axis_name="c", subcore_axis_name="s"),
        scratch_types=(
            [pltpu.VMEM((BLK,), jnp.int32)] * 2
            + [pltpu.VMEM((D * BLK,), jnp.int32)] * 2
            + [pltpu.VMEM((D * BLK,), jnp.float32)] * 2
            + [pltpu.VMEM((BLK,), jnp.float32)] * 8
            + [pltpu.VMEM((D, BLK), jnp.float32)] * 8
            + [pltpu.SemaphoreType.DMA] * 4
        ),
    )
    o_s, o_i, o_j, o_k = run(xt, embf, ws, wi, wj, wk)

    def untranspose(o):
        return o.reshape(f, D, b).transpose(2, 0, 1)

    return (untranspose(o_s), untranspose(o_i),
            untranspose(o_j), untranspose(o_k))
